# baseline (speedup 1.0000x reference)
import functools

import jax
import jax.numpy as jnp
from jax import lax
from jax.experimental import pallas as pl
from jax.experimental.pallas import tpu as pltpu
from jax.experimental.pallas import tpu_sc as plsc

L = 16
NS = 16
NC = 2
NW = NC * NS
_MAGIC = 2.0 ** 23


def _build_sc_call(B, C, H, W, O):
    pairs_per_w = (B * O) // NW
    elems_per_w = pairs_per_w * C
    n_chunks = pairs_per_w // L
    rows_per_chunk = C * L

    mesh = plsc.VectorSubcoreMesh(
        core_axis_name="c", subcore_axis_name="s", num_cores=NC, num_subcores=NS
    )

    @functools.partial(
        pl.kernel,
        out_type=(
            jax.ShapeDtypeStruct((NW * L,), jnp.float32),
            jax.ShapeDtypeStruct((NW * L,), jnp.float32),
        ),
        mesh=mesh,
        compiler_params=pltpu.CompilerParams(
            needs_layout_passes=False, skip_device_barrier=True),
        scratch_types=[
            pltpu.VMEM((pairs_per_w,), jnp.float32),
            pltpu.VMEM((pairs_per_w,), jnp.float32),
            pltpu.VMEM((pairs_per_w,), jnp.int32),
            pltpu.VMEM((pairs_per_w,), jnp.float32),
            pltpu.VMEM((pairs_per_w,), jnp.int32),
            pltpu.VMEM((elems_per_w,), jnp.float32),
            pltpu.VMEM((n_chunks, rows_per_chunk), jnp.int32),
            [pltpu.VMEM((rows_per_chunk, W), jnp.float32)
             for _ in range(n_chunks)],
            pltpu.VMEM((L,), jnp.float32),
            pltpu.VMEM((L,), jnp.float32),
            pltpu.SemaphoreType.DMA,
            pltpu.SemaphoreType.DMA,
        ],
    )
    def sc_call(map_hbm, u_hbm, v_hbm, vis_hbm, s_hbm, psum_hbm, pcnt_hbm,
                u_v, v_v, vis_v, m_v, col_v, s_v, row_v, bufs,
                stage_sum, stage_cnt, sem, sem_in):
        wid = lax.axis_index("s") * NC + lax.axis_index("c")
        base = wid * pairs_per_w

        in_cps = [
            pltpu.async_copy(u_hbm.at[pl.ds(base, pairs_per_w)], u_v, sem_in),
            pltpu.async_copy(v_hbm.at[pl.ds(base, pairs_per_w)], v_v, sem_in),
            pltpu.async_copy(vis_hbm.at[pl.ds(base, pairs_per_w)], vis_v,
                             sem_in),
            pltpu.async_copy(s_hbm.at[pl.ds(base * C, elems_per_w)], s_v,
                             sem_in),
        ]
        for cp in in_cps:
            cp.wait()

        row_base = (base // O) * (C * H)
        wf = jnp.float32(W)
        hf = jnp.float32(H)
        for k in range(n_chunks):
            uu = u_v[pl.ds(k * L, L)]
            vv = v_v[pl.ds(k * L, L)]
            x_f = (uu * wf + _MAGIC) - _MAGIC
            y_f = (vv * hf + _MAGIC) - _MAGIC
            x_i = x_f.astype(jnp.int32)
            y_i = y_f.astype(jnp.int32)
            inb = ((x_i >= 0) & (x_i < W)) & ((y_i >= 0) & (y_i < H))
            visb = vis_v[pl.ds(k * L, L)] != 0
            m = inb & visb
            y_i = jnp.where(m, y_i, 0)
            m_v[pl.ds(k * L, L)] = jnp.where(m, jnp.float32(1.0),
                                             jnp.float32(0.0))
            col_v[pl.ds(k * L, L)] = jnp.where(m, x_i, 0)
            for c in range(C):
                row_v[k, pl.ds(c * L, L)] = y_i + (row_base + c * H)

        cps = [
            pltpu.async_copy(map_hbm.at[row_v.at[k]], bufs[k], sem)
            for k in range(n_chunks)
        ]

        acc = jnp.zeros((L,), jnp.float32)
        cnt = jnp.zeros((L,), jnp.float32)
        lanes = lax.iota(jnp.int32, L)
        for k in range(n_chunks):
            cps[k].wait()
            mk = m_v[pl.ds(k * L, L)]
            cols = col_v[pl.ds(k * L, L)]
            cnt = cnt + mk
            for c in range(C):
                g16 = plsc.load_gather(bufs[k], [lanes + c * L, cols])
                s16 = s_v[pl.ds(c * pairs_per_w + k * L, L)]
                acc = acc + jnp.abs(g16 - s16) * mk

        stage_sum[...] = acc
        stage_cnt[...] = cnt
        out_cps = [
            pltpu.async_copy(stage_sum, psum_hbm.at[pl.ds(wid * L, L)],
                             sem_in),
            pltpu.async_copy(stage_cnt, pcnt_hbm.at[pl.ds(wid * L, L)],
                             sem_in),
        ]
        for cp in out_cps:
            cp.wait()

    return sc_call


def _reduce_body(psum_ref, pcnt_ref, out_ref, C):
    total = jnp.sum(psum_ref[...])
    cnt = jnp.sum(pcnt_ref[...])
    denom = jnp.maximum(cnt * jnp.float32(C), jnp.float32(1.0))
    out_ref[...] = jnp.full(out_ref.shape, total / denom, jnp.float32)


def kernel(scale_map, visibility, keypoint_2d, scale):
    B, C, H, W = scale_map.shape
    O = visibility.shape[1]
    map_rows = scale_map.reshape(B * C * H, W)
    cu = keypoint_2d[:, :, 0, 0].reshape(-1)
    cv = keypoint_2d[:, :, 0, 1].reshape(-1)
    vis = visibility.reshape(-1)
    ppw = (B * O) // NW
    scale_t = jnp.transpose(
        scale.reshape(B * O // ppw, ppw, C), (0, 2, 1)).reshape(-1)
    sc_call = _build_sc_call(B, C, H, W, O)
    psum, pcnt = sc_call(map_rows, cu, cv, vis, scale_t)

    out = pl.pallas_call(
        functools.partial(_reduce_body, C=C),
        out_shape=jax.ShapeDtypeStruct((8, 128), jnp.float32),
    )(psum, pcnt)
    return out[0, 0]


# speedup vs baseline: 1.0411x; 1.0411x over previous
import functools

import jax
import jax.numpy as jnp
from jax import lax
from jax.experimental import pallas as pl
from jax.experimental.pallas import tpu as pltpu
from jax.experimental.pallas import tpu_sc as plsc

L = 16
NS = 16
NC = 1
NW = NC * NS
_MAGIC = 2.0 ** 23


def _build_sc_call(B, C, H, W, O):
    pairs_per_w = (B * O) // NW
    elems_per_w = pairs_per_w * C
    n_chunks = pairs_per_w // L
    rows_per_chunk = C * L

    mesh = plsc.VectorSubcoreMesh(
        core_axis_name="c", subcore_axis_name="s", num_cores=NC, num_subcores=NS
    )

    @functools.partial(
        pl.kernel,
        out_type=(
            jax.ShapeDtypeStruct((NW * L,), jnp.float32),
            jax.ShapeDtypeStruct((NW * L,), jnp.float32),
        ),
        mesh=mesh,
        compiler_params=pltpu.CompilerParams(
            needs_layout_passes=False, skip_device_barrier=True),
        scratch_types=[
            pltpu.VMEM((pairs_per_w,), jnp.float32),
            pltpu.VMEM((pairs_per_w,), jnp.float32),
            pltpu.VMEM((pairs_per_w,), jnp.int32),
            pltpu.VMEM((pairs_per_w,), jnp.float32),
            pltpu.VMEM((pairs_per_w,), jnp.int32),
            pltpu.VMEM((elems_per_w,), jnp.float32),
            pltpu.VMEM((n_chunks, rows_per_chunk), jnp.int32),
            [pltpu.VMEM((rows_per_chunk, W), jnp.float32)
             for _ in range(n_chunks)],
            pltpu.VMEM((L,), jnp.float32),
            pltpu.VMEM((L,), jnp.float32),
            pltpu.SemaphoreType.DMA,
            pltpu.SemaphoreType.DMA,
        ],
    )
    def sc_call(map_hbm, u_hbm, v_hbm, vis_hbm, s_hbm, psum_hbm, pcnt_hbm,
                u_v, v_v, vis_v, m_v, col_v, s_v, row_v, bufs,
                stage_sum, stage_cnt, sem, sem_in):
        wid = lax.axis_index("s") * NC + lax.axis_index("c")
        base = wid * pairs_per_w

        in_cps = [
            pltpu.async_copy(u_hbm.at[pl.ds(base, pairs_per_w)], u_v, sem_in),
            pltpu.async_copy(v_hbm.at[pl.ds(base, pairs_per_w)], v_v, sem_in),
            pltpu.async_copy(vis_hbm.at[pl.ds(base, pairs_per_w)], vis_v,
                             sem_in),
        ]
        s_cp = pltpu.async_copy(
            s_hbm.at[pl.ds(base * C, elems_per_w)], s_v, sem_in)
        for cp in in_cps:
            cp.wait()

        row_base = (base // O) * (C * H)
        wf = jnp.float32(W)
        hf = jnp.float32(H)
        for k in range(n_chunks):
            uu = u_v[pl.ds(k * L, L)]
            vv = v_v[pl.ds(k * L, L)]
            x_f = (uu * wf + _MAGIC) - _MAGIC
            y_f = (vv * hf + _MAGIC) - _MAGIC
            x_i = x_f.astype(jnp.int32)
            y_i = y_f.astype(jnp.int32)
            inb = ((x_i >= 0) & (x_i < W)) & ((y_i >= 0) & (y_i < H))
            visb = vis_v[pl.ds(k * L, L)] != 0
            m = inb & visb
            y_i = jnp.where(m, y_i, 0)
            m_v[pl.ds(k * L, L)] = jnp.where(m, jnp.float32(1.0),
                                             jnp.float32(0.0))
            col_v[pl.ds(k * L, L)] = jnp.where(m, x_i, 0)
            for c in range(C):
                row_v[k, pl.ds(c * L, L)] = y_i + (row_base + c * H)

        cps = [
            pltpu.async_copy(map_hbm.at[row_v.at[k]], bufs[k], sem)
            for k in range(n_chunks)
        ]

        s_cp.wait()
        acc = jnp.zeros((L,), jnp.float32)
        cnt = jnp.zeros((L,), jnp.float32)
        lanes = lax.iota(jnp.int32, L)
        for k in range(n_chunks):
            cps[k].wait()
            mk = m_v[pl.ds(k * L, L)]
            cols = col_v[pl.ds(k * L, L)]
            cnt = cnt + mk
            for c in range(C):
                g16 = plsc.load_gather(bufs[k], [lanes + c * L, cols])
                s16 = s_v[pl.ds(c * pairs_per_w + k * L, L)]
                acc = acc + jnp.abs(g16 - s16) * mk

        stage_sum[...] = acc
        stage_cnt[...] = cnt
        out_cps = [
            pltpu.async_copy(stage_sum, psum_hbm.at[pl.ds(wid * L, L)],
                             sem_in),
            pltpu.async_copy(stage_cnt, pcnt_hbm.at[pl.ds(wid * L, L)],
                             sem_in),
        ]
        for cp in out_cps:
            cp.wait()

    return sc_call


def _reduce_body(psum_ref, pcnt_ref, out_ref, C):
    total = jnp.sum(psum_ref[...])
    cnt = jnp.sum(pcnt_ref[...])
    denom = jnp.maximum(cnt * jnp.float32(C), jnp.float32(1.0))
    out_ref[...] = jnp.full(out_ref.shape, total / denom, jnp.float32)


def kernel(scale_map, visibility, keypoint_2d, scale):
    B, C, H, W = scale_map.shape
    O = visibility.shape[1]
    map_rows = scale_map.reshape(B * C * H, W)
    cu = keypoint_2d[:, :, 0, 0].reshape(-1)
    cv = keypoint_2d[:, :, 0, 1].reshape(-1)
    vis = visibility.reshape(-1)
    ppw = (B * O) // NW
    scale_t = jnp.transpose(
        scale.reshape(B * O // ppw, ppw, C), (0, 2, 1)).reshape(-1)
    sc_call = _build_sc_call(B, C, H, W, O)
    psum, pcnt = sc_call(map_rows, cu, cv, vis, scale_t)

    out = pl.pallas_call(
        functools.partial(_reduce_body, C=C),
        out_shape=jax.ShapeDtypeStruct((8, 128), jnp.float32),
    )(psum, pcnt)
    return out[0, 0]
